# Initial kernel scaffold; baseline (speedup 1.0000x reference)
import jax
import jax.numpy as jnp
from jax.experimental import pallas as pl


def kernel(x, adj_norm, weight, bias):
    raise NotImplementedError("write your pallas kernel here")


# speedup vs baseline: 1.0003x; 1.0003x over previous
import jax
import jax.numpy as jnp
from jax.experimental import pallas as pl
from jax.experimental.pallas import tpu as pltpu

_BM = 400


def _support_body(x_ref, w_ref, s_ref):
    s_ref[...] = jnp.dot(
        x_ref[...].astype(jnp.bfloat16),
        w_ref[...].astype(jnp.bfloat16),
        preferred_element_type=jnp.float32,
    ).astype(jnp.bfloat16)


def _agg_body(adj_ref, s_ref, b_ref, o_ref):
    o_ref[...] = (
        jnp.dot(
            adj_ref[...].astype(jnp.bfloat16),
            s_ref[...],
            preferred_element_type=jnp.float32,
        )
        + b_ref[...]
    )


def kernel(x, adj_norm, weight, bias):
    n, d_in = x.shape
    d_out = weight.shape[1]

    support = pl.pallas_call(
        _support_body,
        out_shape=jax.ShapeDtypeStruct((n, d_out), jnp.bfloat16),
    )(x, weight)

    out = pl.pallas_call(
        _agg_body,
        grid=(n // _BM,),
        in_specs=[
            pl.BlockSpec((_BM, n), lambda m: (m, 0)),
            pl.BlockSpec((n, d_out), lambda m: (0, 0)),
            pl.BlockSpec((1, d_out), lambda m: (0, 0)),
        ],
        out_specs=pl.BlockSpec((_BM, d_out), lambda m: (m, 0)),
        out_shape=jax.ShapeDtypeStruct((n, d_out), jnp.float32),
        compiler_params=pltpu.CompilerParams(
            dimension_semantics=("arbitrary",),
        ),
    )(adj_norm, support, bias.reshape(1, d_out))
    return out


# speedup vs baseline: 1.0174x; 1.0170x over previous
import jax
import jax.numpy as jnp
from jax.experimental import pallas as pl
from jax.experimental.pallas import tpu as pltpu

_BM = 400


def _body(x_ref, w_ref, adj_ref, b_ref, o_ref, s_ref):
    m = pl.program_id(0)

    @pl.when(m == 0)
    def _support():
        s_ref[...] = jnp.dot(
            x_ref[...].astype(jnp.bfloat16),
            w_ref[...].astype(jnp.bfloat16),
            preferred_element_type=jnp.float32,
        ).astype(jnp.bfloat16)

    @pl.when(m > 0)
    def _aggregate():
        o_ref[...] = (
            jnp.dot(
                adj_ref[...].astype(jnp.bfloat16),
                s_ref[...],
                preferred_element_type=jnp.float32,
            )
            + b_ref[...]
        )


def kernel(x, adj_norm, weight, bias):
    n, d_in = x.shape
    d_out = weight.shape[1]

    return pl.pallas_call(
        _body,
        grid=(1 + n // _BM,),
        in_specs=[
            pl.BlockSpec((n, d_in), lambda m: (0, 0)),
            pl.BlockSpec((d_in, d_out), lambda m: (0, 0)),
            pl.BlockSpec((_BM, n), lambda m: (jnp.maximum(m - 1, 0), 0)),
            pl.BlockSpec((1, d_out), lambda m: (0, 0)),
        ],
        out_specs=pl.BlockSpec((_BM, d_out), lambda m: (jnp.maximum(m - 1, 0), 0)),
        out_shape=jax.ShapeDtypeStruct((n, d_out), jnp.float32),
        scratch_shapes=[pltpu.VMEM((n, d_out), jnp.bfloat16)],
        compiler_params=pltpu.CompilerParams(
            dimension_semantics=("arbitrary",),
        ),
    )(x, weight, adj_norm, bias.reshape(1, d_out))
